# baseline (speedup 1.0000x reference)
import jax
import jax.numpy as jnp
import numpy as np
from jax import lax
from jax.experimental import pallas as pl
from jax.experimental.pallas import tpu as pltpu
from jax.experimental.pallas import tpu_sc as plsc

G = 1000
SIZE = 0.001
DELTAZ = 0.001
COEF_A = 0.0136
COEF_B = 0.038
SQRT12 = np.sqrt(12.0)

LANE = 128
R = 8192
NPAD = R * LANE

NC, NS = 2, 16
NW = NC * NS
GSZ = 128
HALVES = 4
HROWS = R // HALVES
WROWS = HROWS // NW

TSTRIPE = 31256
TPAD = TSTRIPE * NS


def _sc_gather_body(idx_hbm, tab_hbm, out_hbm, idxbuf, gbuf, bounce, shared, sem_g):
    sid = lax.axis_index("s")
    wid = sid * NC + lax.axis_index("c")
    rbase = wid * WROWS
    toff = sid * TSTRIPE
    pltpu.sync_copy(tab_hbm.at[pl.ds(toff, TSTRIPE)], bounce)
    pltpu.sync_copy(bounce, shared.at[pl.ds(toff, TSTRIPE)])
    pltpu.sync_copy(idx_hbm.at[pl.ds(rbase, WROWS)], idxbuf)
    plsc.subcore_barrier()

    def fire(k, c):
        pltpu.async_copy(shared.at[idxbuf.at[k]], gbuf.at[k], sem_g)
        return c

    lax.fori_loop(0, WROWS, fire, 0)
    pltpu.make_async_copy(out_hbm.at[pl.ds(rbase, WROWS)], gbuf, sem_g).wait()
    pltpu.sync_copy(gbuf, out_hbm.at[pl.ds(rbase, WROWS)])


def _sc_gather(widx2d, table_packed):
    mesh = plsc.VectorSubcoreMesh(core_axis_name="c", subcore_axis_name="s")
    k = pl.kernel(
        _sc_gather_body,
        out_type=jax.ShapeDtypeStruct((HROWS, GSZ), jnp.int32),
        mesh=mesh,
        scratch_types=[
            pltpu.VMEM((WROWS, GSZ), jnp.int32),
            pltpu.VMEM((WROWS, GSZ), jnp.int32),
            pltpu.VMEM((TSTRIPE,), jnp.int32),
            pltpu.VMEM_SHARED((TPAD,), jnp.int32),
            pltpu.SemaphoreType.DMA,
        ],
    )
    return k(widx2d, table_packed)


def _voxel_lin(x, y):
    ix = jnp.clip((x / SIZE).astype(jnp.int32), 0, G - 1)
    iy = jnp.clip((y / SIZE).astype(jnp.int32), 0, G - 1)
    return ix * G + iy


def _tc_idx_body(xy_ref, idx_ref):
    lin = _voxel_lin(xy_ref[0], xy_ref[1])
    idx_ref[...] = lin - jnp.where(lin >= TPAD, TPAD, 0)


def _tc_idx(xyT):
    BR = 1024
    return pl.pallas_call(
        _tc_idx_body,
        grid=(R // BR,),
        in_specs=[pl.BlockSpec((2, BR, LANE), lambda i: (0, i, 0))],
        out_specs=pl.BlockSpec((BR, LANE), lambda i: (i, 0)),
        out_shape=jax.ShapeDtypeStruct((R, LANE), jnp.int32),
        compiler_params=pltpu.CompilerParams(
            dimension_semantics=("arbitrary",),
        ),
    )(xyT)


def _tc_physics_body(xy_ref, tt_ref, z1_ref, z2_ref, mom_ref, w_ref, out_ref):
    x = xy_ref[0]
    y = xy_ref[1]
    mask = (x >= 0.0) & (x < 1.0) & (y >= 0.0) & (y < 1.0)
    lin = _voxel_lin(x, y)
    sh = jnp.where(lin >= TPAD, 16, 0)
    bits = lax.shift_left(lax.shift_right_arithmetic(w_ref[...], sh), 16)
    x0 = lax.bitcast_convert_type(bits, jnp.float32)
    mom = mom_ref[...]
    a_over_p = COEF_A / mom
    for c in (0, 1):
        t = tt_ref[c]
        z1 = z1_ref[c]
        z2 = z2_ref[c]
        cos_t = jnp.cos(t)
        flight = DELTAZ / cos_t
        n_x0 = flight / x0
        theta0 = a_over_p * jnp.sqrt(n_x0)
        theta0 = theta0 * (1.0 + COEF_B * jnp.log(n_x0))
        dtheta = z1 * theta0
        dxy = flight * jnp.sin(theta0) * (z1 / SQRT12 + z2 / 2.0)
        dxy_vol = dxy * cos_t
        out_ref[c] = jnp.where(mask, dtheta, 0.0)
        out_ref[c + 2] = jnp.where(mask, dxy_vol, 0.0)


def _tc_physics(xyT, ttT, z1T, z2T, momp, words, half):
    BR = 512
    grid = (HROWS // BR,)
    off = half * (HROWS // BR)
    pair = pl.BlockSpec((2, BR, LANE), lambda i: (0, i + off, 0))
    single = pl.BlockSpec((BR, LANE), lambda i: (i + off, 0))
    wspec = pl.BlockSpec((BR, LANE), lambda i: (i, 0))
    return pl.pallas_call(
        _tc_physics_body,
        grid=grid,
        in_specs=[pair, pair, pair, pair, single, wspec],
        out_specs=pl.BlockSpec((4, BR, LANE), lambda i: (0, i, 0)),
        out_shape=jax.ShapeDtypeStruct((4, HROWS, LANE), jnp.float32),
        compiler_params=pltpu.CompilerParams(
            dimension_semantics=("arbitrary",),
        ),
    )(xyT, ttT, z1T, z2T, momp, words)


def _pack_table(rad_length):
    t = jnp.pad(rad_length.reshape(G * G), (0, 2 * TPAD - G * G))
    u = lax.bitcast_convert_type(t, jnp.int32)
    one = jnp.int32(1)
    rnd = u + jnp.int32(0x7FFF) + jnp.bitwise_and(lax.shift_right_logical(u, 16), one)
    hi16 = jnp.bitwise_and(rnd, jnp.int32(-65536))
    lo = lax.shift_right_logical(hi16[:TPAD], 16)
    return jnp.bitwise_or(lo, hi16[TPAD:])


def kernel(xy, theta_xy, mom, z1, z2, rad_length):
    n = xy.shape[0]
    pad = NPAD - n

    def prep_pair(a):
        return jnp.pad(a, ((0, pad), (0, 0))).T.reshape(2, R, LANE)

    xyT = prep_pair(xy)
    ttT = prep_pair(theta_xy)
    z1T = prep_pair(z1)
    z2T = prep_pair(z2)
    momp = jnp.pad(mom, (0, pad)).reshape(R, LANE)

    widx = _tc_idx(xyT)
    table = _pack_table(rad_length)

    nh = NPAD // HALVES
    parts = []
    for h in range(HALVES):
        words = _sc_gather(widx[h * HROWS:(h + 1) * HROWS], table)
        out4 = _tc_physics(xyT, ttT, z1T, z2T, momp, words, h)
        lo = h * nh
        take = min(max(n - lo, 0), nh)
        parts.append(out4.reshape(4, nh)[:, :take].T)
    return jnp.concatenate(parts, axis=0)


# speedup vs baseline: 1.2141x; 1.2141x over previous
import jax
import jax.numpy as jnp
import numpy as np
from jax import lax
from jax.experimental import pallas as pl
from jax.experimental.pallas import tpu as pltpu
from jax.experimental.pallas import tpu_sc as plsc

G = 1000
SIZE = 0.001
DELTAZ = 0.001
COEF_A = 0.0136
COEF_B = 0.038
SQRT12 = np.sqrt(12.0)

LANE = 128
R = 8192
NPAD = R * LANE

NC, NS = 2, 16
NW = NC * NS
GSZ = 128
HALVES = 2
HROWS = R // HALVES
WROWS = HROWS // NW

TSTRIPE = 31256
TPAD = TSTRIPE * NS


def _sc_gather_body(idx_hbm, tab_hbm, out_hbm, idxbuf, gbuf, bounce, shared, sem_g, sem_i):
    sid = lax.axis_index("s")
    wid = sid * NC + lax.axis_index("c")
    rbase = wid * WROWS
    pltpu.async_copy(idx_hbm.at[pl.ds(rbase, WROWS)], idxbuf, sem_i)
    toff = sid * TSTRIPE
    pltpu.sync_copy(tab_hbm.at[pl.ds(toff, TSTRIPE)], bounce)
    pltpu.sync_copy(bounce, shared.at[pl.ds(toff, TSTRIPE)])
    pltpu.make_async_copy(idx_hbm.at[pl.ds(rbase, WROWS)], idxbuf, sem_i).wait()
    plsc.subcore_barrier()

    def fire(k, c):
        pltpu.async_copy(shared.at[idxbuf.at[k]], gbuf.at[k], sem_g)
        return c

    lax.fori_loop(0, WROWS, fire, 0)
    pltpu.make_async_copy(out_hbm.at[pl.ds(rbase, WROWS)], gbuf, sem_g).wait()
    pltpu.sync_copy(gbuf, out_hbm.at[pl.ds(rbase, WROWS)])


def _sc_gather(widx2d, table_packed):
    mesh = plsc.VectorSubcoreMesh(core_axis_name="c", subcore_axis_name="s")
    k = pl.kernel(
        _sc_gather_body,
        out_type=jax.ShapeDtypeStruct((HROWS, GSZ), jnp.int32),
        mesh=mesh,
        scratch_types=[
            pltpu.VMEM((WROWS, GSZ), jnp.int32),
            pltpu.VMEM((WROWS, GSZ), jnp.int32),
            pltpu.VMEM((TSTRIPE,), jnp.int32),
            pltpu.VMEM_SHARED((TPAD,), jnp.int32),
            pltpu.SemaphoreType.DMA,
            pltpu.SemaphoreType.DMA,
        ],
    )
    return k(widx2d, table_packed)


def _voxel_lin(x, y):
    ix = jnp.clip((x / SIZE).astype(jnp.int32), 0, G - 1)
    iy = jnp.clip((y / SIZE).astype(jnp.int32), 0, G - 1)
    return ix * G + iy


def _tc_idx_body(xy_ref, idx_ref):
    lin = _voxel_lin(xy_ref[0], xy_ref[1])
    idx_ref[...] = lin - jnp.where(lin >= TPAD, TPAD, 0)


def _tc_idx(xyT):
    BR = 1024
    return pl.pallas_call(
        _tc_idx_body,
        grid=(R // BR,),
        in_specs=[pl.BlockSpec((2, BR, LANE), lambda i: (0, i, 0))],
        out_specs=pl.BlockSpec((BR, LANE), lambda i: (i, 0)),
        out_shape=jax.ShapeDtypeStruct((R, LANE), jnp.int32),
        compiler_params=pltpu.CompilerParams(
            dimension_semantics=("arbitrary",),
        ),
    )(xyT)


def _tc_physics_body(xy_ref, tt_ref, z1_ref, z2_ref, mom_ref, w_ref, out_ref):
    x = xy_ref[0]
    y = xy_ref[1]
    mask = (x >= 0.0) & (x < 1.0) & (y >= 0.0) & (y < 1.0)
    lin = _voxel_lin(x, y)
    sh = jnp.where(lin >= TPAD, 16, 0)
    bits = lax.shift_left(lax.shift_right_arithmetic(w_ref[...], sh), 16)
    x0 = lax.bitcast_convert_type(bits, jnp.float32)
    mom = mom_ref[...]
    a_over_p = COEF_A / mom
    for c in (0, 1):
        t = tt_ref[c]
        z1 = z1_ref[c]
        z2 = z2_ref[c]
        cos_t = jnp.cos(t)
        flight = DELTAZ / cos_t
        n_x0 = flight / x0
        theta0 = a_over_p * jnp.sqrt(n_x0)
        theta0 = theta0 * (1.0 + COEF_B * jnp.log(n_x0))
        dtheta = z1 * theta0
        dxy = flight * jnp.sin(theta0) * (z1 / SQRT12 + z2 / 2.0)
        dxy_vol = dxy * cos_t
        out_ref[c] = jnp.where(mask, dtheta, 0.0)
        out_ref[c + 2] = jnp.where(mask, dxy_vol, 0.0)


def _tc_physics(xyT, ttT, z1T, z2T, momp, words, half):
    BR = 512
    grid = (HROWS // BR,)
    off = half * (HROWS // BR)
    pair = pl.BlockSpec((2, BR, LANE), lambda i: (0, i + off, 0))
    single = pl.BlockSpec((BR, LANE), lambda i: (i + off, 0))
    wspec = pl.BlockSpec((BR, LANE), lambda i: (i, 0))
    return pl.pallas_call(
        _tc_physics_body,
        grid=grid,
        in_specs=[pair, pair, pair, pair, single, wspec],
        out_specs=pl.BlockSpec((4, BR, LANE), lambda i: (0, i, 0)),
        out_shape=jax.ShapeDtypeStruct((4, HROWS, LANE), jnp.float32),
        compiler_params=pltpu.CompilerParams(
            dimension_semantics=("arbitrary",),
        ),
    )(xyT, ttT, z1T, z2T, momp, words)


def _pack_table(rad_length):
    t = jnp.pad(rad_length.reshape(G * G), (0, 2 * TPAD - G * G))
    u = lax.bitcast_convert_type(t, jnp.int32)
    one = jnp.int32(1)
    rnd = u + jnp.int32(0x7FFF) + jnp.bitwise_and(lax.shift_right_logical(u, 16), one)
    hi16 = jnp.bitwise_and(rnd, jnp.int32(-65536))
    lo = lax.shift_right_logical(hi16[:TPAD], 16)
    return jnp.bitwise_or(lo, hi16[TPAD:])


def kernel(xy, theta_xy, mom, z1, z2, rad_length):
    n = xy.shape[0]
    pad = NPAD - n

    def prep_pair(a):
        return jnp.pad(a, ((0, pad), (0, 0))).T.reshape(2, R, LANE)

    xyT = prep_pair(xy)
    ttT = prep_pair(theta_xy)
    z1T = prep_pair(z1)
    z2T = prep_pair(z2)
    momp = jnp.pad(mom, (0, pad)).reshape(R, LANE)

    widx = _tc_idx(xyT)
    table = _pack_table(rad_length)

    nh = NPAD // HALVES
    parts = []
    for h in range(HALVES):
        words = _sc_gather(widx[h * HROWS:(h + 1) * HROWS], table)
        out4 = _tc_physics(xyT, ttT, z1T, z2T, momp, words, h)
        lo = h * nh
        take = min(max(n - lo, 0), nh)
        parts.append(out4.reshape(4, nh)[:, :take].T)
    return jnp.concatenate(parts, axis=0)


# speedup vs baseline: 1.2566x; 1.0350x over previous
import jax
import jax.numpy as jnp
import numpy as np
from jax import lax
from jax.experimental import pallas as pl
from jax.experimental.pallas import tpu as pltpu
from jax.experimental.pallas import tpu_sc as plsc

G = 1000
SIZE = 0.001
DELTAZ = 0.001
COEF_A = 0.0136
COEF_B = 0.038
SQRT12 = np.sqrt(12.0)

LANE = 128
R = 8192
NPAD = R * LANE

NC, NS = 2, 16
NW = NC * NS
GSZ = 128
HALVES = 2
HROWS = R // HALVES
WROWS = HROWS // NW

TSTRIPE = 31256
TPAD = TSTRIPE * NS


def _sc_gather_body(idx_hbm, tab_hbm, out_hbm, idxbuf, gbuf, bounce, shared, sem_g, sem_i):
    sid = lax.axis_index("s")
    wid = sid * NC + lax.axis_index("c")
    rbase = wid * WROWS
    pltpu.async_copy(idx_hbm.at[pl.ds(rbase, WROWS)], idxbuf, sem_i)
    toff = sid * TSTRIPE
    pltpu.sync_copy(tab_hbm.at[pl.ds(toff, TSTRIPE)], bounce)
    pltpu.sync_copy(bounce, shared.at[pl.ds(toff, TSTRIPE)])
    pltpu.make_async_copy(idx_hbm.at[pl.ds(rbase, WROWS)], idxbuf, sem_i).wait()
    plsc.subcore_barrier()

    def fire(k, c):
        pltpu.async_copy(shared.at[idxbuf.at[k]], gbuf.at[k], sem_g)
        return c

    lax.fori_loop(0, WROWS, fire, 0)
    pltpu.make_async_copy(out_hbm.at[pl.ds(rbase, WROWS)], gbuf, sem_g).wait()
    pltpu.sync_copy(gbuf, out_hbm.at[pl.ds(rbase, WROWS)])


def _sc_gather(widx2d, table_packed):
    mesh = plsc.VectorSubcoreMesh(core_axis_name="c", subcore_axis_name="s")
    k = pl.kernel(
        _sc_gather_body,
        out_type=jax.ShapeDtypeStruct((HROWS, GSZ), jnp.int32),
        mesh=mesh,
        scratch_types=[
            pltpu.VMEM((WROWS, GSZ), jnp.int32),
            pltpu.VMEM((WROWS, GSZ), jnp.int32),
            pltpu.VMEM((TSTRIPE,), jnp.int32),
            pltpu.VMEM_SHARED((TPAD,), jnp.int32),
            pltpu.SemaphoreType.DMA,
            pltpu.SemaphoreType.DMA,
        ],
    )
    return k(widx2d, table_packed)


def _voxel_lin(x, y):
    ix = jnp.clip((x / SIZE).astype(jnp.int32), 0, G - 1)
    iy = jnp.clip((y / SIZE).astype(jnp.int32), 0, G - 1)
    return ix * G + iy


def _tc_idx_body(xy_ref, idx_ref):
    lin = _voxel_lin(xy_ref[0], xy_ref[1])
    idx_ref[...] = lin - jnp.where(lin >= TPAD, TPAD, 0)


def _tc_idx(xyT, half):
    BR = 1024
    off = half * (HROWS // BR)
    return pl.pallas_call(
        _tc_idx_body,
        grid=(HROWS // BR,),
        in_specs=[pl.BlockSpec((2, BR, LANE), lambda i: (0, i + off, 0))],
        out_specs=pl.BlockSpec((BR, LANE), lambda i: (i, 0)),
        out_shape=jax.ShapeDtypeStruct((HROWS, LANE), jnp.int32),
        compiler_params=pltpu.CompilerParams(
            dimension_semantics=("arbitrary",),
        ),
    )(xyT)


def _tc_physics_body(xy_ref, tt_ref, z1_ref, z2_ref, mom_ref, w_ref, out_ref):
    x = xy_ref[0]
    y = xy_ref[1]
    mask = (x >= 0.0) & (x < 1.0) & (y >= 0.0) & (y < 1.0)
    lin = _voxel_lin(x, y)
    sh = jnp.where(lin >= TPAD, 16, 0)
    bits = lax.shift_left(lax.shift_right_arithmetic(w_ref[...], sh), 16)
    x0 = lax.bitcast_convert_type(bits, jnp.float32)
    mom = mom_ref[...]
    a_over_p = COEF_A / mom
    for c in (0, 1):
        t = tt_ref[c]
        z1 = z1_ref[c]
        z2 = z2_ref[c]
        cos_t = jnp.cos(t)
        flight = DELTAZ / cos_t
        n_x0 = flight / x0
        theta0 = a_over_p * jnp.sqrt(n_x0)
        theta0 = theta0 * (1.0 + COEF_B * jnp.log(n_x0))
        dtheta = z1 * theta0
        dxy = flight * jnp.sin(theta0) * (z1 / SQRT12 + z2 / 2.0)
        dxy_vol = dxy * cos_t
        out_ref[c] = jnp.where(mask, dtheta, 0.0)
        out_ref[c + 2] = jnp.where(mask, dxy_vol, 0.0)


def _tc_physics(xyT, ttT, z1T, z2T, momp, words, half):
    BR = 512
    grid = (HROWS // BR,)
    off = half * (HROWS // BR)
    pair = pl.BlockSpec((2, BR, LANE), lambda i: (0, i + off, 0))
    single = pl.BlockSpec((BR, LANE), lambda i: (i + off, 0))
    wspec = pl.BlockSpec((BR, LANE), lambda i: (i, 0))
    return pl.pallas_call(
        _tc_physics_body,
        grid=grid,
        in_specs=[pair, pair, pair, pair, single, wspec],
        out_specs=pl.BlockSpec((4, BR, LANE), lambda i: (0, i, 0)),
        out_shape=jax.ShapeDtypeStruct((4, HROWS, LANE), jnp.float32),
        compiler_params=pltpu.CompilerParams(
            dimension_semantics=("arbitrary",),
        ),
    )(xyT, ttT, z1T, z2T, momp, words)


def _pack_table(rad_length):
    t = jnp.pad(rad_length.reshape(G * G), (0, 2 * TPAD - G * G))
    u = lax.bitcast_convert_type(t, jnp.int32)
    one = jnp.int32(1)
    rnd = u + jnp.int32(0x7FFF) + jnp.bitwise_and(lax.shift_right_logical(u, 16), one)
    hi16 = jnp.bitwise_and(rnd, jnp.int32(-65536))
    lo = lax.shift_right_logical(hi16[:TPAD], 16)
    return jnp.bitwise_or(lo, hi16[TPAD:])


def kernel(xy, theta_xy, mom, z1, z2, rad_length):
    n = xy.shape[0]
    pad = NPAD - n

    def prep_pair(a):
        return jnp.pad(a, ((0, pad), (0, 0))).T.reshape(2, R, LANE)

    xyT = prep_pair(xy)
    ttT = prep_pair(theta_xy)
    z1T = prep_pair(z1)
    z2T = prep_pair(z2)
    momp = jnp.pad(mom, (0, pad)).reshape(R, LANE)

    table = _pack_table(rad_length)

    nh = NPAD // HALVES
    parts = []
    for h in range(HALVES):
        words = _sc_gather(_tc_idx(xyT, h), table)
        out4 = _tc_physics(xyT, ttT, z1T, z2T, momp, words, h)
        lo = h * nh
        take = min(max(n - lo, 0), nh)
        parts.append(out4.reshape(4, nh)[:, :take].T)
    return jnp.concatenate(parts, axis=0)
